# baseline (device time: 25549 ns/iter reference)
import jax
import jax.numpy as jnp
from jax import lax
from jax.experimental import pallas as pl
from jax.experimental.pallas import tpu as pltpu

N_DEV = 4


def kernel(x):
    m_per, n = x.shape
    h = m_per // 2

    def body(x_ref, out_ref, comm_ref, send_sems, recv_sems, copy_sems):
        my = lax.axis_index("i")
        left = (my - 1) % N_DEV
        right = (my + 1) % N_DEV
        opp = (my + 2) % N_DEV

        barrier_sem = pltpu.get_barrier_semaphore()
        for nbr in [left, right]:
            pl.semaphore_signal(
                barrier_sem, inc=1,
                device_id=(nbr,), device_id_type=pl.DeviceIdType.MESH,
            )
        pl.semaphore_wait(barrier_sem, 2)

        def send(idx, src, dst, dev):
            return pltpu.make_async_remote_copy(
                src_ref=src, dst_ref=dst,
                send_sem=send_sems.at[idx], recv_sem=recv_sems.at[idx],
                device_id=(dev,), device_id_type=pl.DeviceIdType.MESH,
            )

        a = send(0, x_ref.at[pl.ds(0, h)], comm_ref.at[my, pl.ds(0, h)], right)
        b = send(1, x_ref.at[pl.ds(h, h)], comm_ref.at[my, pl.ds(h, h)], right)
        d = send(3, x_ref.at[pl.ds(h, h)], comm_ref.at[my, pl.ds(h, h)], left)
        e = send(4, x_ref.at[pl.ds(0, h)], comm_ref.at[my, pl.ds(0, h)], left)
        a.start()
        b.start()
        d.start()
        e.start()

        copy_own = pltpu.make_async_copy(
            x_ref, out_ref.at[pl.ds(my * m_per, m_per)], copy_sems.at[0]
        )
        copy_own.start()

        def recv(idx, dst):
            return pltpu.make_async_remote_copy(
                src_ref=dst, dst_ref=dst,
                send_sem=send_sems.at[idx], recv_sem=recv_sems.at[idx],
                device_id=(my,), device_id_type=pl.DeviceIdType.MESH,
            )

        recv_a = recv(0, comm_ref.at[left, pl.ds(0, h)])
        recv_b = recv(1, comm_ref.at[left, pl.ds(h, h)])
        recv_d = recv(3, comm_ref.at[right, pl.ds(h, h)])
        recv_e = recv(4, comm_ref.at[right, pl.ds(0, h)])
        recv_c = recv(2, comm_ref.at[opp, pl.ds(0, h)])
        recv_f = recv(5, comm_ref.at[opp, pl.ds(h, h)])

        recv_a.wait_recv()
        b.wait_send()
        c = send(2, comm_ref.at[left, pl.ds(0, h)],
                 comm_ref.at[left, pl.ds(0, h)], right)
        c.start()
        recv_d.wait_recv()
        e.wait_send()
        f = send(5, comm_ref.at[right, pl.ds(h, h)],
                 comm_ref.at[right, pl.ds(h, h)], left)
        f.start()

        recv_b.wait_recv()
        cp_left = pltpu.make_async_copy(
            comm_ref.at[left], out_ref.at[pl.ds(left * m_per, m_per)],
            copy_sems.at[1],
        )
        cp_left.start()
        recv_e.wait_recv()
        cp_right = pltpu.make_async_copy(
            comm_ref.at[right], out_ref.at[pl.ds(right * m_per, m_per)],
            copy_sems.at[2],
        )
        cp_right.start()
        recv_c.wait_recv()
        cp_opp_a = pltpu.make_async_copy(
            comm_ref.at[opp, pl.ds(0, h)],
            out_ref.at[pl.ds(opp * m_per, h)],
            copy_sems.at[3],
        )
        cp_opp_a.start()
        recv_f.wait_recv()
        cp_opp_b = pltpu.make_async_copy(
            comm_ref.at[opp, pl.ds(h, h)],
            out_ref.at[pl.ds(opp * m_per + h, h)],
            copy_sems.at[4],
        )
        cp_opp_b.start()

        for r in (a, c, d, f):
            r.wait_send()
        for cp in (copy_own, cp_left, cp_right, cp_opp_a, cp_opp_b):
            cp.wait()

    return pl.pallas_call(
        body,
        out_shape=jax.ShapeDtypeStruct((N_DEV * m_per, n), x.dtype),
        in_specs=[pl.BlockSpec(memory_space=pltpu.VMEM)],
        out_specs=pl.BlockSpec(memory_space=pltpu.MemorySpace.HBM),
        scratch_shapes=[
            pltpu.VMEM((N_DEV, m_per, n), x.dtype),
            pltpu.SemaphoreType.DMA((6,)),
            pltpu.SemaphoreType.DMA((6,)),
            pltpu.SemaphoreType.DMA((5,)),
        ],
        compiler_params=pltpu.CompilerParams(collective_id=0),
    )(x)


# device time: 23799 ns/iter; 1.0735x vs baseline; 1.0735x over previous
import jax
import jax.numpy as jnp
from jax import lax
from jax.experimental import pallas as pl
from jax.experimental.pallas import tpu as pltpu

N_DEV = 4


def kernel(x):
    m_per, n = x.shape
    h = m_per // 2

    def body(x_ref, out_ref, send_sems, recv_sems, copy_sem):
        my = lax.axis_index("i")
        left = (my - 1) % N_DEV
        right = (my + 1) % N_DEV
        opp = (my + 2) % N_DEV

        barrier_sem = pltpu.get_barrier_semaphore()
        for nbr in [left, right]:
            pl.semaphore_signal(
                barrier_sem, inc=1,
                device_id=(nbr,), device_id_type=pl.DeviceIdType.MESH,
            )
        pl.semaphore_wait(barrier_sem, 2)

        def send(idx, src, dst, dev):
            return pltpu.make_async_remote_copy(
                src_ref=src, dst_ref=dst,
                send_sem=send_sems.at[idx], recv_sem=recv_sems.at[idx],
                device_id=(dev,), device_id_type=pl.DeviceIdType.MESH,
            )

        a = send(0, x_ref.at[pl.ds(0, h)],
                 out_ref.at[pl.ds(my * m_per, h)], right)
        b = send(1, x_ref.at[pl.ds(h, h)],
                 out_ref.at[pl.ds(my * m_per + h, h)], right)
        d = send(3, x_ref.at[pl.ds(h, h)],
                 out_ref.at[pl.ds(my * m_per + h, h)], left)
        e = send(4, x_ref.at[pl.ds(0, h)],
                 out_ref.at[pl.ds(my * m_per, h)], left)
        a.start()
        b.start()
        d.start()
        e.start()

        copy_own = pltpu.make_async_copy(
            x_ref, out_ref.at[pl.ds(my * m_per, m_per)], copy_sem
        )
        copy_own.start()

        def recv(idx, dst):
            return pltpu.make_async_remote_copy(
                src_ref=dst, dst_ref=dst,
                send_sem=send_sems.at[idx], recv_sem=recv_sems.at[idx],
                device_id=(my,), device_id_type=pl.DeviceIdType.MESH,
            )

        recv_a = recv(0, out_ref.at[pl.ds(left * m_per, h)])
        recv_b = recv(1, out_ref.at[pl.ds(left * m_per + h, h)])
        recv_d = recv(3, out_ref.at[pl.ds(right * m_per + h, h)])
        recv_e = recv(4, out_ref.at[pl.ds(right * m_per, h)])
        recv_c = recv(2, out_ref.at[pl.ds(opp * m_per, h)])
        recv_f = recv(5, out_ref.at[pl.ds(opp * m_per + h, h)])

        recv_a.wait_recv()
        c = send(2, out_ref.at[pl.ds(left * m_per, h)],
                 out_ref.at[pl.ds(left * m_per, h)], right)
        c.start()
        recv_d.wait_recv()
        f = send(5, out_ref.at[pl.ds(right * m_per + h, h)],
                 out_ref.at[pl.ds(right * m_per + h, h)], left)
        f.start()

        recv_b.wait_recv()
        recv_e.wait_recv()
        recv_c.wait_recv()
        recv_f.wait_recv()
        for r in (a, b, c, d, e, f):
            r.wait_send()
        copy_own.wait()

    return pl.pallas_call(
        body,
        out_shape=jax.ShapeDtypeStruct((N_DEV * m_per, n), x.dtype),
        in_specs=[pl.BlockSpec(memory_space=pl.ANY)],
        out_specs=pl.BlockSpec(memory_space=pltpu.VMEM),
        scratch_shapes=[
            pltpu.SemaphoreType.DMA((6,)),
            pltpu.SemaphoreType.DMA((6,)),
            pltpu.SemaphoreType.DMA,
        ],
        compiler_params=pltpu.CompilerParams(collective_id=0),
    )(x)
